# baseline (speedup 1.0000x reference)
import functools

import jax
import jax.numpy as jnp
from jax import lax
from jax.experimental import pallas as pl
from jax.experimental.pallas import tpu as pltpu
from jax.experimental.pallas import tpu_sc as plsc

_B = 128
_V = 32128
_K = 64
_L = 16
_NC = 2
_NS = 16
_NW = _NC * _NS
_NCH = _V // _L

_B_SC = 32
_B_TC = _B - _B_SC
_BLK = 16


def _tc_body(x_ref, o_ref):
    sign = jnp.int32(-0x80000000)
    low31 = jnp.int32(0x7FFFFFFF)
    x = x_ref[...]
    bits = jax.lax.bitcast_convert_type(x, jnp.int32)
    mkey = jnp.where(bits < 0, bits ^ low31, bits)

    def step(i, uprefix):
        bit = lax.shift_left(jnp.int32(1), jnp.int32(31) - i)
        ut = uprefix | bit
        st = ut ^ sign
        cnt = jnp.sum((mkey >= st).astype(jnp.int32), axis=1, keepdims=True)
        return jnp.where(cnt >= _K, ut, uprefix)

    uprefix = lax.fori_loop(0, 32, step, jnp.zeros((_BLK, 1), jnp.int32))
    kkey = uprefix ^ sign
    fbits = jnp.where(kkey < 0, kkey ^ low31, kkey)
    kth = jax.lax.bitcast_convert_type(fbits, jnp.float32)

    m = jnp.max(x, axis=1, keepdims=True)
    e = jnp.where(x < kth, 0.0, jnp.exp(x - m))
    z = jnp.sum(e, axis=1, keepdims=True)
    o_ref[...] = e * (1.0 / z)


def _vec(s):
    return jnp.broadcast_to(s, (_L,))


def _splat_reduce(x, op, tmp):
    for sh in (1, 2, 4, 8):
        tmp[pl.ds(0, _L)] = x
        tmp[pl.ds(_L, _L)] = x
        x = op(x, tmp[pl.ds(sh, _L)])
    return x


def _sc_body(x_hbm, key_hbm, out_hbm, row_v, key_v, tmp_f, tmp_i):
    wid = lax.axis_index("s") * _NC + lax.axis_index("c")

    i32_zero = jnp.zeros((_L,), jnp.int32)
    f32_zero = jnp.zeros((_L,), jnp.float32)
    f32_one = jnp.ones((_L,), jnp.float32)
    i32_one = jnp.ones((_L,), jnp.int32)
    sign_v = jnp.full((_L,), -0x80000000, jnp.int32)
    k_v = jnp.full((_L,), _K, jnp.int32)
    neg_inf = jnp.full((_L,), -jnp.inf, jnp.float32)

    pltpu.sync_copy(x_hbm.at[wid], row_v)
    pltpu.sync_copy(key_hbm.at[wid], key_v)

    def p1(j, gm):
        g0, g1, g2, g3 = gm
        b = j * (4 * _L)
        g0 = jnp.maximum(g0, row_v[pl.ds(b, _L)])
        g1 = jnp.maximum(g1, row_v[pl.ds(b + _L, _L)])
        g2 = jnp.maximum(g2, row_v[pl.ds(b + 2 * _L, _L)])
        g3 = jnp.maximum(g3, row_v[pl.ds(b + 3 * _L, _L)])
        return (g0, g1, g2, g3)

    g0, g1, g2, g3 = lax.fori_loop(0, _NCH // 4, p1,
                                   (neg_inf, neg_inf, neg_inf, neg_inf))
    hi2 = jnp.maximum(jnp.maximum(g0, g1), jnp.maximum(g2, g3))
    m = _splat_reduce(hi2, jnp.maximum, tmp_f)

    def p3(i, uprefix):
        bit = lax.shift_left(i32_one, _vec(jnp.int32(31) - i))
        ut = uprefix | bit
        st = ut ^ sign_v

        def cnt_body(j, accs):
            a0, a1 = accs
            b = j * (2 * _L)
            v0 = key_v[pl.ds(b, _L)]
            v1 = key_v[pl.ds(b + _L, _L)]
            a0 = a0 + jnp.where(v0 >= st, i32_one, i32_zero)
            a1 = a1 + jnp.where(v1 >= st, i32_one, i32_zero)
            return (a0, a1)

        a0, a1 = lax.fori_loop(0, _NCH // 2, cnt_body, (i32_zero, i32_zero))
        cnt = _splat_reduce(a0 + a1, jnp.add, tmp_i)
        return jnp.where(cnt >= k_v, ut, uprefix)

    uprefix = lax.fori_loop(0, 32, p3, i32_zero)
    kth_key = uprefix ^ sign_v

    def p4(j, zacc):
        b = j * _L
        v = row_v[pl.ds(b, _L)]
        kv = key_v[pl.ds(b, _L)]
        e = jnp.where(kv >= kth_key, jnp.exp(v - m), f32_zero)
        row_v[pl.ds(b, _L)] = e
        return zacc + e

    zacc = lax.fori_loop(0, _NCH, p4, f32_zero)
    rz = f32_one / _splat_reduce(zacc, jnp.add, tmp_f)

    def p5(j, carry):
        b = j * _L
        row_v[pl.ds(b, _L)] = row_v[pl.ds(b, _L)] * rz
        return carry

    lax.fori_loop(0, _NCH, p5, jnp.int32(0))
    pltpu.sync_copy(row_v, out_hbm.at[wid])


@jax.jit
def kernel(next_logits, k):
    del k

    x_tc = next_logits[:_B_TC]
    x_sc = next_logits[_B_TC:]
    bits_sc = jax.lax.bitcast_convert_type(x_sc, jnp.int32)
    keys_sc = jnp.where(bits_sc < 0, bits_sc ^ jnp.int32(0x7FFFFFFF), bits_sc)

    out_tc = pl.pallas_call(
        _tc_body,
        out_shape=jax.ShapeDtypeStruct((_B_TC, _V), jnp.float32),
        grid=(_B_TC // _BLK,),
        in_specs=[pl.BlockSpec((_BLK, _V), lambda i: (i, 0))],
        out_specs=pl.BlockSpec((_BLK, _V), lambda i: (i, 0)),
    )(x_tc)

    mesh = plsc.VectorSubcoreMesh(core_axis_name="c", subcore_axis_name="s")
    sc_fn = functools.partial(
        pl.kernel,
        mesh=mesh,
        out_type=jax.ShapeDtypeStruct((_B_SC, _V), jnp.float32),
        scratch_types=[
            pltpu.VMEM((_V,), jnp.float32),
            pltpu.VMEM((_V,), jnp.int32),
            pltpu.VMEM((2 * _L,), jnp.float32),
            pltpu.VMEM((2 * _L,), jnp.int32),
        ],
    )(_sc_body)
    out_sc = sc_fn(x_sc, keys_sc)

    return jnp.concatenate([out_tc, out_sc], axis=0)


# speedup vs baseline: 1.6089x; 1.6089x over previous
import functools

import jax
import jax.numpy as jnp
from jax import lax
from jax.experimental import pallas as pl
from jax.experimental.pallas import tpu as pltpu
from jax.experimental.pallas import tpu_sc as plsc

_B = 128
_V = 32128
_K = 64
_L = 16
_NC = 2
_NS = 16
_NW = _NC * _NS
_NCH = _V // _L

_B_SC = 32
_B_TC = _B - _B_SC
_BLK = 16


def _tc_body(x_ref, o_ref):
    sign = jnp.int32(-0x80000000)
    low31 = jnp.int32(0x7FFFFFFF)
    x = x_ref[...]
    bits = jax.lax.bitcast_convert_type(x, jnp.int32)
    mkey = jnp.where(bits < 0, bits ^ low31, bits)

    def step(i, uprefix):
        bit = lax.shift_left(jnp.int32(1), jnp.int32(31) - i)
        ut = uprefix | bit
        st = ut ^ sign
        cnt = jnp.sum((mkey >= st).astype(jnp.int32), axis=1, keepdims=True)
        return jnp.where(cnt >= _K, ut, uprefix)

    uprefix = lax.fori_loop(0, 32, step, jnp.zeros((_BLK, 1), jnp.int32))
    kkey = uprefix ^ sign
    fbits = jnp.where(kkey < 0, kkey ^ low31, kkey)
    kth = jax.lax.bitcast_convert_type(fbits, jnp.float32)

    m = jnp.max(x, axis=1, keepdims=True)
    e = jnp.where(x < kth, 0.0, jnp.exp(x - m))
    z = jnp.sum(e, axis=1, keepdims=True)
    o_ref[...] = e * (1.0 / z)


def _vec(s):
    return jnp.broadcast_to(s, (_L,))


def _splat_reduce(x, op, tmp):
    for sh in (1, 2, 4, 8):
        tmp[pl.ds(0, _L)] = x
        tmp[pl.ds(_L, _L)] = x
        x = op(x, tmp[pl.ds(sh, _L)])
    return x


def _sc_body(x_hbm, key_hbm, out_hbm, row_v, key_v, tmp_f, tmp_i):
    wid = lax.axis_index("s") * _NC + lax.axis_index("c")

    i32_zero = jnp.zeros((_L,), jnp.int32)
    f32_zero = jnp.zeros((_L,), jnp.float32)
    f32_one = jnp.ones((_L,), jnp.float32)
    i32_one = jnp.ones((_L,), jnp.int32)
    sign_v = jnp.full((_L,), -0x80000000, jnp.int32)
    k_v = jnp.full((_L,), _K, jnp.int32)
    neg_inf = jnp.full((_L,), -jnp.inf, jnp.float32)

    pltpu.sync_copy(x_hbm.at[wid], row_v)
    pltpu.sync_copy(key_hbm.at[wid], key_v)

    min_key = jnp.full((_L,), -0x80000000, jnp.int32)

    @plsc.parallel_loop(0, _NCH, step=4,
                        carry=(neg_inf, neg_inf, min_key, min_key,
                               min_key, min_key))
    def p1(j, gm):
        f0, f1, k0, k1, k2, k3 = gm
        b = j * _L
        f0 = jnp.maximum(f0, row_v[pl.ds(b, _L)])
        f1 = jnp.maximum(f1, row_v[pl.ds(b + 2 * _L, _L)])
        f0 = jnp.maximum(f0, row_v[pl.ds(b + _L, _L)])
        f1 = jnp.maximum(f1, row_v[pl.ds(b + 3 * _L, _L)])
        k0 = jnp.maximum(k0, key_v[pl.ds(b, _L)])
        k1 = jnp.maximum(k1, key_v[pl.ds(b + _L, _L)])
        k2 = jnp.maximum(k2, key_v[pl.ds(b + 2 * _L, _L)])
        k3 = jnp.maximum(k3, key_v[pl.ds(b + 3 * _L, _L)])
        return (f0, f1, k0, k1, k2, k3)

    f0, f1, k0, k1, k2, k3 = p1
    m = _splat_reduce(jnp.maximum(f0, f1), jnp.maximum, tmp_f)
    lo2 = jnp.minimum(jnp.minimum(k0, k1), jnp.minimum(k2, k3))
    hi2 = jnp.maximum(jnp.maximum(k0, k1), jnp.maximum(k2, k3))
    t0k = _splat_reduce(lo2, jnp.minimum, tmp_i)
    mxk = _splat_reduce(hi2, jnp.maximum, tmp_i)

    ut0 = t0k ^ sign_v
    umx = mxk ^ sign_v
    d = ut0 ^ umx
    d = d | lax.shift_right_logical(d, _vec(jnp.int32(1)))
    d = d | lax.shift_right_logical(d, _vec(jnp.int32(2)))
    d = d | lax.shift_right_logical(d, _vec(jnp.int32(4)))
    d = d | lax.shift_right_logical(d, _vec(jnp.int32(8)))
    d = d | lax.shift_right_logical(d, _vec(jnp.int32(16)))
    prefix0 = umx & ~d
    bit0 = lax.shift_right_logical(d, _vec(jnp.int32(1))) + i32_one
    pc = d - (lax.shift_right_logical(d, _vec(jnp.int32(1)))
              & jnp.full((_L,), 0x55555555, jnp.int32))
    pc = ((pc & jnp.full((_L,), 0x33333333, jnp.int32)) +
          (lax.shift_right_logical(pc, _vec(jnp.int32(2)))
           & jnp.full((_L,), 0x33333333, jnp.int32)))
    pc = ((pc + lax.shift_right_logical(pc, _vec(jnp.int32(4))))
          & jnp.full((_L,), 0x0F0F0F0F, jnp.int32))
    pc = lax.shift_right_logical(pc * jnp.full((_L,), 0x01010101, jnp.int32),
                                 _vec(jnp.int32(24)))
    n_it = pc[0]

    def srch_body(i, c):
        uprefix, bitv = c
        ut = uprefix | bitv
        st = ut ^ sign_v

        @plsc.parallel_loop(0, _NCH, step=8,
                            carry=(i32_zero, i32_zero, i32_zero, i32_zero))
        def cnt(j, accs):
            a0, a1, a2, a3 = accs
            b = j * _L
            a0 = a0 + jnp.where(key_v[pl.ds(b, _L)] >= st, i32_one, i32_zero)
            a1 = a1 + jnp.where(key_v[pl.ds(b + _L, _L)] >= st, i32_one, i32_zero)
            a2 = a2 + jnp.where(key_v[pl.ds(b + 2 * _L, _L)] >= st, i32_one, i32_zero)
            a3 = a3 + jnp.where(key_v[pl.ds(b + 3 * _L, _L)] >= st, i32_one, i32_zero)
            a0 = a0 + jnp.where(key_v[pl.ds(b + 4 * _L, _L)] >= st, i32_one, i32_zero)
            a1 = a1 + jnp.where(key_v[pl.ds(b + 5 * _L, _L)] >= st, i32_one, i32_zero)
            a2 = a2 + jnp.where(key_v[pl.ds(b + 6 * _L, _L)] >= st, i32_one, i32_zero)
            a3 = a3 + jnp.where(key_v[pl.ds(b + 7 * _L, _L)] >= st, i32_one, i32_zero)
            return (a0, a1, a2, a3)

        a0, a1, a2, a3 = cnt
        total = _splat_reduce((a0 + a1) + (a2 + a3), jnp.add, tmp_i)
        uprefix = jnp.where(total >= k_v, ut, uprefix)
        return (uprefix, lax.shift_right_logical(bitv, _vec(jnp.int32(1))))

    uprefix, _ = lax.fori_loop(0, n_it, srch_body, (prefix0, bit0))
    kth_key = uprefix ^ sign_v

    @plsc.parallel_loop(0, _NCH, step=2, carry=(f32_zero, f32_zero))
    def p4(j, zaccs):
        z0, z1 = zaccs
        b = j * _L
        v0 = row_v[pl.ds(b, _L)]
        v1 = row_v[pl.ds(b + _L, _L)]
        kv0 = key_v[pl.ds(b, _L)]
        kv1 = key_v[pl.ds(b + _L, _L)]
        e0 = jnp.where(kv0 >= kth_key, jnp.exp(v0 - m), f32_zero)
        e1 = jnp.where(kv1 >= kth_key, jnp.exp(v1 - m), f32_zero)
        row_v[pl.ds(b, _L)] = e0
        row_v[pl.ds(b + _L, _L)] = e1
        return (z0 + e0, z1 + e1)

    z0, z1 = p4
    rz = f32_one / _splat_reduce(z0 + z1, jnp.add, tmp_f)

    @plsc.parallel_loop(0, _NCH, step=4)
    def p5(j):
        b = j * _L
        row_v[pl.ds(b, _L)] = row_v[pl.ds(b, _L)] * rz
        row_v[pl.ds(b + _L, _L)] = row_v[pl.ds(b + _L, _L)] * rz
        row_v[pl.ds(b + 2 * _L, _L)] = row_v[pl.ds(b + 2 * _L, _L)] * rz
        row_v[pl.ds(b + 3 * _L, _L)] = row_v[pl.ds(b + 3 * _L, _L)] * rz

    pltpu.sync_copy(row_v, out_hbm.at[wid])


@jax.jit
def kernel(next_logits, k):
    del k

    x_tc = next_logits[:_B_TC]
    x_sc = next_logits[_B_TC:]
    bits_sc = jax.lax.bitcast_convert_type(x_sc, jnp.int32)
    keys_sc = jnp.where(bits_sc < 0, bits_sc ^ jnp.int32(0x7FFFFFFF), bits_sc)

    out_tc = pl.pallas_call(
        _tc_body,
        out_shape=jax.ShapeDtypeStruct((_B_TC, _V), jnp.float32),
        grid=(_B_TC // _BLK,),
        in_specs=[pl.BlockSpec((_BLK, _V), lambda i: (i, 0))],
        out_specs=pl.BlockSpec((_BLK, _V), lambda i: (i, 0)),
    )(x_tc)

    mesh = plsc.VectorSubcoreMesh(core_axis_name="c", subcore_axis_name="s")
    sc_fn = functools.partial(
        pl.kernel,
        mesh=mesh,
        out_type=jax.ShapeDtypeStruct((_B_SC, _V), jnp.float32),
        scratch_types=[
            pltpu.VMEM((_V,), jnp.float32),
            pltpu.VMEM((_V,), jnp.int32),
            pltpu.VMEM((2 * _L,), jnp.float32),
            pltpu.VMEM((2 * _L,), jnp.int32),
        ],
    )(_sc_body)
    out_sc = sc_fn(x_sc, keys_sc)

    return jnp.concatenate([out_tc, out_sc], axis=0)


# speedup vs baseline: 1.9756x; 1.2279x over previous
import functools

import jax
import jax.numpy as jnp
from jax import lax
from jax.experimental import pallas as pl

_B = 128
_V = 32128
_K = 64
_BLK = 16
_NT = _V // 128


def _body(x_ref, o_ref):
    sign = jnp.int32(-0x80000000)
    low31 = jnp.int32(0x7FFFFFFF)
    x = x_ref[...]

    cm = x[:, 0:128]
    for i in range(1, _NT):
        cm = jnp.maximum(cm, x[:, i * 128:(i + 1) * 128])
    cmb = lax.bitcast_convert_type(cm, jnp.int32)
    cmk = jnp.where(cmb < 0, cmb ^ low31, cmb)

    def mini(i, up):
        bit = lax.shift_left(jnp.int32(1), jnp.int32(31) - i)
        ut = up | bit
        st = ut ^ sign
        c = jnp.sum((cmk >= st).astype(jnp.int32), axis=1, keepdims=True)
        return jnp.where(c >= _K, ut, up)

    ut0 = lax.fori_loop(0, 32, mini, jnp.zeros((_BLK, 1), jnp.int32))

    maxkey = jnp.max(cmk, axis=1, keepdims=True)
    umax = maxkey ^ sign
    mfb = jnp.where(maxkey < 0, maxkey ^ low31, maxkey)
    m = lax.bitcast_convert_type(mfb, jnp.float32)

    d = ut0 ^ umax
    d = d | lax.shift_right_logical(d, 1)
    d = d | lax.shift_right_logical(d, 2)
    d = d | lax.shift_right_logical(d, 4)
    d = d | lax.shift_right_logical(d, 8)
    d = d | lax.shift_right_logical(d, 16)
    prefix0 = umax & ~d
    bit0 = lax.shift_right_logical(d, 1) + 1
    pc = d - (lax.shift_right_logical(d, 1) & jnp.int32(0x55555555))
    pc = ((pc & jnp.int32(0x33333333)) +
          (lax.shift_right_logical(pc, 2) & jnp.int32(0x33333333)))
    pc = (pc + lax.shift_right_logical(pc, 4)) & jnp.int32(0x0F0F0F0F)
    pc = lax.shift_right_logical(pc * jnp.int32(0x01010101), 24)
    n_it = jnp.max(pc)

    def step(i, carry):
        up, bitv = carry
        ut = up | bitv
        st = ut ^ sign
        fb = jnp.where(st < 0, st ^ low31, st)
        tf = lax.bitcast_convert_type(fb, jnp.float32)
        cnt = jnp.sum((x >= tf).astype(jnp.int32), axis=1, keepdims=True)
        up = jnp.where(cnt >= _K, ut, up)
        return (up, lax.shift_right_logical(bitv, 1))

    up, _ = lax.fori_loop(0, n_it, step, (prefix0, bit0))
    kkey = (up ^ sign)
    fbits = jnp.where(kkey < 0, kkey ^ low31, kkey)
    kth = lax.bitcast_convert_type(fbits, jnp.float32)

    e = jnp.where(x < kth, 0.0, jnp.exp(x - m))
    z = jnp.sum(e, axis=1, keepdims=True)
    o_ref[...] = e * (1.0 / z)


@jax.jit
def kernel(next_logits, k):
    del k
    return pl.pallas_call(
        _body,
        out_shape=jax.ShapeDtypeStruct((_B, _V), jnp.float32),
        grid=(_B // _BLK,),
        in_specs=[pl.BlockSpec((_BLK, _V), lambda i: (i, 0))],
        out_specs=pl.BlockSpec((_BLK, _V), lambda i: (i, 0)),
    )(next_logits)


# speedup vs baseline: 2.4505x; 1.2404x over previous
import functools

import jax
import jax.numpy as jnp
from jax import lax
from jax.experimental import pallas as pl

_B = 128
_V = 32128
_K = 64
_BLK = 32
_NT = _V // 128


def _body(x_ref, o_ref):
    sign = jnp.int32(-0x80000000)
    low31 = jnp.int32(0x7FFFFFFF)
    x = x_ref[...]

    cm = x[:, 0:128]
    for i in range(1, _NT):
        cm = jnp.maximum(cm, x[:, i * 128:(i + 1) * 128])
    cmb = lax.bitcast_convert_type(cm, jnp.int32)
    cmk = jnp.where(cmb < 0, cmb ^ low31, cmb)

    def mini(i, up):
        bit = lax.shift_left(jnp.int32(1), jnp.int32(31) - i)
        ut = up | bit
        st = ut ^ sign
        c = jnp.sum((cmk >= st).astype(jnp.int32), axis=1, keepdims=True)
        return jnp.where(c >= _K, ut, up)

    ut0 = lax.fori_loop(0, 32, mini, jnp.zeros((_BLK, 1), jnp.int32))

    maxkey = jnp.max(cmk, axis=1, keepdims=True)
    umax = maxkey ^ sign
    mfb = jnp.where(maxkey < 0, maxkey ^ low31, maxkey)
    m = lax.bitcast_convert_type(mfb, jnp.float32)

    d = ut0 ^ umax
    d = d | lax.shift_right_logical(d, 1)
    d = d | lax.shift_right_logical(d, 2)
    d = d | lax.shift_right_logical(d, 4)
    d = d | lax.shift_right_logical(d, 8)
    d = d | lax.shift_right_logical(d, 16)
    prefix0 = umax & ~d
    bit0 = lax.shift_right_logical(d, 1) + 1
    pc = d - (lax.shift_right_logical(d, 1) & jnp.int32(0x55555555))
    pc = ((pc & jnp.int32(0x33333333)) +
          (lax.shift_right_logical(pc, 2) & jnp.int32(0x33333333)))
    pc = (pc + lax.shift_right_logical(pc, 4)) & jnp.int32(0x0F0F0F0F)
    pc = lax.shift_right_logical(pc * jnp.int32(0x01010101), 24)
    n_it = jnp.max(pc)

    def step(i, carry):
        up, bitv = carry
        ut = up | bitv
        st = ut ^ sign
        fb = jnp.where(st < 0, st ^ low31, st)
        tf = lax.bitcast_convert_type(fb, jnp.float32)
        cnt = jnp.sum((x >= tf).astype(jnp.int32), axis=1, keepdims=True)
        up = jnp.where(cnt >= _K, ut, up)
        return (up, lax.shift_right_logical(bitv, 1))

    up, _ = lax.fori_loop(0, n_it, step, (prefix0, bit0))
    kkey = (up ^ sign)
    fbits = jnp.where(kkey < 0, kkey ^ low31, kkey)
    kth = lax.bitcast_convert_type(fbits, jnp.float32)

    e = jnp.where(x < kth, 0.0, jnp.exp(x - m))
    z = jnp.sum(e, axis=1, keepdims=True)
    o_ref[...] = e * (1.0 / z)


@jax.jit
def kernel(next_logits, k):
    del k
    return pl.pallas_call(
        _body,
        out_shape=jax.ShapeDtypeStruct((_B, _V), jnp.float32),
        grid=(_B // _BLK,),
        in_specs=[pl.BlockSpec((_BLK, _V), lambda i: (i, 0))],
        out_specs=pl.BlockSpec((_BLK, _V), lambda i: (i, 0)),
    )(next_logits)


# speedup vs baseline: 3.0491x; 1.2443x over previous
import functools

import jax
import jax.numpy as jnp
from jax import lax
from jax.experimental import pallas as pl

_B = 128
_V = 32128
_K = 64
_BLK = 64
_NT = _V // 128


def _body(x_ref, o_ref):
    sign = jnp.int32(-0x80000000)
    low31 = jnp.int32(0x7FFFFFFF)
    x = x_ref[...]

    cm = x[:, 0:128]
    for i in range(1, _NT):
        cm = jnp.maximum(cm, x[:, i * 128:(i + 1) * 128])
    cmb = lax.bitcast_convert_type(cm, jnp.int32)
    cmk = jnp.where(cmb < 0, cmb ^ low31, cmb)

    def mini(i, up):
        bit = lax.shift_left(jnp.int32(1), jnp.int32(31) - i)
        ut = up | bit
        st = ut ^ sign
        c = jnp.sum((cmk >= st).astype(jnp.int32), axis=1, keepdims=True)
        return jnp.where(c >= _K, ut, up)

    ut0 = lax.fori_loop(0, 32, mini, jnp.zeros((_BLK, 1), jnp.int32))

    maxkey = jnp.max(cmk, axis=1, keepdims=True)
    umax = maxkey ^ sign
    mfb = jnp.where(maxkey < 0, maxkey ^ low31, maxkey)
    m = lax.bitcast_convert_type(mfb, jnp.float32)

    d = ut0 ^ umax
    d = d | lax.shift_right_logical(d, 1)
    d = d | lax.shift_right_logical(d, 2)
    d = d | lax.shift_right_logical(d, 4)
    d = d | lax.shift_right_logical(d, 8)
    d = d | lax.shift_right_logical(d, 16)
    prefix0 = umax & ~d
    bit0 = lax.shift_right_logical(d, 1) + 1
    pc = d - (lax.shift_right_logical(d, 1) & jnp.int32(0x55555555))
    pc = ((pc & jnp.int32(0x33333333)) +
          (lax.shift_right_logical(pc, 2) & jnp.int32(0x33333333)))
    pc = (pc + lax.shift_right_logical(pc, 4)) & jnp.int32(0x0F0F0F0F)
    pc = lax.shift_right_logical(pc * jnp.int32(0x01010101), 24)
    n_it = jnp.max(pc)

    def step(i, carry):
        up, bitv = carry
        ut = up | bitv
        st = ut ^ sign
        fb = jnp.where(st < 0, st ^ low31, st)
        tf = lax.bitcast_convert_type(fb, jnp.float32)
        cnt = jnp.sum((x >= tf).astype(jnp.int32), axis=1, keepdims=True)
        up = jnp.where(cnt >= _K, ut, up)
        return (up, lax.shift_right_logical(bitv, 1))

    up, _ = lax.fori_loop(0, n_it, step, (prefix0, bit0))
    kkey = (up ^ sign)
    fbits = jnp.where(kkey < 0, kkey ^ low31, kkey)
    kth = lax.bitcast_convert_type(fbits, jnp.float32)

    e = jnp.where(x < kth, 0.0, jnp.exp(x - m))
    z = jnp.sum(e, axis=1, keepdims=True)
    o_ref[...] = e * (1.0 / z)


@jax.jit
def kernel(next_logits, k):
    del k
    return pl.pallas_call(
        _body,
        out_shape=jax.ShapeDtypeStruct((_B, _V), jnp.float32),
        grid=(_B // _BLK,),
        in_specs=[pl.BlockSpec((_BLK, _V), lambda i: (i, 0))],
        out_specs=pl.BlockSpec((_BLK, _V), lambda i: (i, 0)),
    )(next_logits)


# speedup vs baseline: 3.1489x; 1.0327x over previous
import functools

import jax
import jax.numpy as jnp
from jax import lax
from jax.experimental import pallas as pl

_B = 128
_V = 32128
_K = 64
_BLK = 128
_NT = _V // 128


def _body(x_ref, o_ref):
    sign = jnp.int32(-0x80000000)
    low31 = jnp.int32(0x7FFFFFFF)
    x = x_ref[...]

    cm = x[:, 0:128]
    for i in range(1, _NT):
        cm = jnp.maximum(cm, x[:, i * 128:(i + 1) * 128])
    cmb = lax.bitcast_convert_type(cm, jnp.int32)
    cmk = jnp.where(cmb < 0, cmb ^ low31, cmb)

    def mini(i, up):
        bit = lax.shift_left(jnp.int32(1), jnp.int32(31) - i)
        ut = up | bit
        st = ut ^ sign
        c = jnp.sum((cmk >= st).astype(jnp.int32), axis=1, keepdims=True)
        return jnp.where(c >= _K, ut, up)

    ut0 = lax.fori_loop(0, 32, mini, jnp.zeros((_BLK, 1), jnp.int32))

    maxkey = jnp.max(cmk, axis=1, keepdims=True)
    umax = maxkey ^ sign
    mfb = jnp.where(maxkey < 0, maxkey ^ low31, maxkey)
    m = lax.bitcast_convert_type(mfb, jnp.float32)

    d = ut0 ^ umax
    d = d | lax.shift_right_logical(d, 1)
    d = d | lax.shift_right_logical(d, 2)
    d = d | lax.shift_right_logical(d, 4)
    d = d | lax.shift_right_logical(d, 8)
    d = d | lax.shift_right_logical(d, 16)
    prefix0 = umax & ~d
    bit0 = lax.shift_right_logical(d, 1) + 1
    pc = d - (lax.shift_right_logical(d, 1) & jnp.int32(0x55555555))
    pc = ((pc & jnp.int32(0x33333333)) +
          (lax.shift_right_logical(pc, 2) & jnp.int32(0x33333333)))
    pc = (pc + lax.shift_right_logical(pc, 4)) & jnp.int32(0x0F0F0F0F)
    pc = lax.shift_right_logical(pc * jnp.int32(0x01010101), 24)
    n_it = jnp.max(pc)

    def step(i, carry):
        up, bitv = carry
        ut = up | bitv
        st = ut ^ sign
        fb = jnp.where(st < 0, st ^ low31, st)
        tf = lax.bitcast_convert_type(fb, jnp.float32)
        cnt = jnp.sum((x >= tf).astype(jnp.int32), axis=1, keepdims=True)
        up = jnp.where(cnt >= _K, ut, up)
        return (up, lax.shift_right_logical(bitv, 1))

    up, _ = lax.fori_loop(0, n_it, step, (prefix0, bit0))
    kkey = (up ^ sign)
    fbits = jnp.where(kkey < 0, kkey ^ low31, kkey)
    kth = lax.bitcast_convert_type(fbits, jnp.float32)

    e = jnp.where(x < kth, 0.0, jnp.exp(x - m))
    z = jnp.sum(e, axis=1, keepdims=True)
    o_ref[...] = e * (1.0 / z)


@jax.jit
def kernel(next_logits, k):
    del k
    return pl.pallas_call(
        _body,
        out_shape=jax.ShapeDtypeStruct((_B, _V), jnp.float32),
        grid=(_B // _BLK,),
        in_specs=[pl.BlockSpec((_BLK, _V), lambda i: (i, 0))],
        out_specs=pl.BlockSpec((_BLK, _V), lambda i: (i, 0)),
    )(next_logits)
